# baseline (speedup 1.0000x reference)
import functools

import jax
import jax.numpy as jnp
from jax import lax
from jax.experimental import pallas as pl
from jax.experimental.pallas import tpu as pltpu
from jax.experimental.pallas import tpu_sc as plsc

K = 32
_NC, _NS, _L = 2, 16, 16
NEG = -3.0e38
BIG = 3.0e38


def _topk_body(xyz_ref, q_ref, meta_ref):
    x = xyz_ref[0]
    y = xyz_ref[1]
    z = xyz_ref[2]
    B, N = x.shape
    iota = jax.lax.broadcasted_iota(jnp.int32, (2 * B, N), 1)
    parts = []
    for q in range(2):
        dx = x - q_ref[q, 0]
        dy = y - q_ref[q, 1]
        dz = z - q_ref[q, 2]
        parts.append(dx * dx + dy * dy + dz * dz)
    d2 = jnp.concatenate(parts, axis=0)
    args = []
    for k in range(K):
        m = jnp.min(d2, axis=1, keepdims=True)
        arg = jnp.min(jnp.where(d2 == m, iota, N), axis=1, keepdims=True)
        a2 = arg.reshape(2, B, 1)
        for q in range(2):
            meta_ref[:, pl.ds(q * K + k, 1)] = a2[q]
        d2 = jnp.where(iota == arg, BIG, d2)
        args.append(a2)
    J = 2 * K
    aj = [args[j % K][j // K] for j in range(J)]
    blk = [a // 128 for a in aj]
    isf = []
    for k in range(J):
        f = jnp.ones_like(blk[0], dtype=jnp.bool_)
        for j in range(k):
            f = jnp.logical_and(f, blk[j] != blk[k])
        isf.append(f)
    uu = []
    run = jnp.zeros_like(blk[0])
    for k in range(J):
        run = run + isf[k].astype(jnp.int32)
        uu.append(run - 1)
    mcnt = run
    for k in range(J):
        pos = jnp.zeros_like(blk[0])
        for j in range(k + 1):
            hit = jnp.logical_and(isf[j], blk[j] == blk[k])
            pos = pos + jnp.where(hit, uu[j], 0)
        meta_ref[:, pl.ds(2 * J + k, 1)] = pos
    for i in range(J):
        u = jnp.zeros_like(blk[0])
        for j in range(J):
            hit = jnp.logical_and(isf[j], uu[j] == i)
            u = u + jnp.where(hit, blk[j], 0)
        meta_ref[:, pl.ds(J + i, 1)] = u
    meta_ref[:, pl.ds(3 * J, _L)] = jnp.broadcast_to(mcnt, (2 * B // 2, _L))


def _topk(points_xyz, joint_origin, drag_point):
    B, N, _ = points_xyz.shape
    xyz_t = jnp.moveaxis(points_xyz, -1, 0)
    q = jnp.stack([joint_origin, drag_point], 0)
    q = jnp.moveaxis(q, -1, 1)[..., None]
    return pl.pallas_call(
        _topk_body,
        out_shape=jax.ShapeDtypeStruct((B, 6 * K + _L), jnp.int32),
    )(xyz_t, q)


def _maxpool_body(idx_ref, pf_ref, out_ref, *, nblk):
    b = pl.program_id(0)
    nb = pl.program_id(1)

    @pl.when(nb == 0)
    def _():
        out_ref[...] = jnp.full_like(out_ref, NEG)

    pf = pf_ref[0]
    niota = jax.lax.broadcasted_iota(jnp.int32, (1, nblk), 1) + nb * nblk
    for q in range(2):
        mask = jnp.zeros((1, nblk), dtype=jnp.bool_)
        for k in range(K):
            mask = jnp.logical_or(mask, niota == idx_ref[q, b, k])
        masked = jnp.where(mask, pf, NEG)
        cur = jnp.max(masked, axis=1)
        out_ref[0, q, :] = jnp.maximum(out_ref[0, q, :], cur)


def _maxpool(point_features, idx):
    B, C, N = point_features.shape
    NBLK = 512
    grid = (B, N // NBLK)
    out = pl.pallas_call(
        functools.partial(_maxpool_body, nblk=NBLK),
        grid=grid,
        in_specs=[
            pl.BlockSpec(memory_space=pltpu.SMEM),
            pl.BlockSpec((1, C, NBLK), lambda b, nb: (b, 0, nb)),
        ],
        out_specs=pl.BlockSpec((1, 2, C), lambda b, nb: (b, 0, 0)),
        out_shape=jax.ShapeDtypeStruct((B, 2, C), jnp.float32),
    )(idx, point_features)
    return out


_CHUNK = 128


_RC = 256


def _sc_body_v4(pf_hbm, meta_hbm, out_hbm, meta_v, buf3, out_buf, sems,
                *, B, C, N):
    J = 2 * K
    CQ = C // 4
    s = lax.axis_index("s")
    c = lax.axis_index("c")
    wid = s * _NC + c
    b = wid // 4
    c0 = (wid % 4) * CQ

    pltpu.sync_copy(meta_hbm.at[b], meta_v)
    iota = lax.iota(jnp.int32, _L)
    nkv = [meta_v[pl.ds(v * _L, _L)] for v in range(J // _L)]
    ubv = [meta_v[pl.ds((J + v * _L), _L)] for v in range(J // _L)]
    pov = [meta_v[pl.ds((2 * J + v * _L), _L)] for v in range(J // _L)]
    mcnt = jnp.max(meta_v[pl.ds(3 * J, _L)])
    rstart = b * C + c0

    lanes, poss = [], []
    for k in range(J):
        sel = iota == (k % _L)
        lanes.append(jnp.max(jnp.where(sel, nkv[k // _L], 0)) % 128)
        poss.append(jnp.max(jnp.where(sel, pov[k // _L], 0)))

    def _iloop(j, _):
        out_buf[pl.ds(j * _L, _L)] = jnp.full((_L,), float(NEG), jnp.float32)
        return 0

    lax.fori_loop(0, 2 * CQ // _L, _iloop, 0)

    T = mcnt

    def _ub(i):
        r = jnp.zeros((), jnp.int32)
        for v in range(J // _L):
            sel = (iota == (i % _L)) & (i >= v * _L) & (i < (v + 1) * _L)
            r = jnp.maximum(r, jnp.max(jnp.where(sel, ubv[v], 0)))
        return r

    def _fire(t):
        n0 = pl.multiple_of(_ub(t) * 128, 128)
        pltpu.make_async_copy(
            pf_hbm.at[pl.ds(rstart, CQ), pl.ds(n0, 128)],
            buf3.at[t % 3], sems.at[t % 3]).start()

    def _consume(t):
        pltpu.make_async_copy(
            pf_hbm.at[pl.ds(rstart, CQ), pl.ds(0, 128)],
            buf3.at[t % 3], sems.at[t % 3]).wait()
        t3 = jnp.broadcast_to(t % 3, (_L,))
        for k in range(J):
            @pl.when(poss[k] == t)
            def _(k=k, t3=t3):
                lane_b = jnp.broadcast_to(lanes[k], (_L,))
                qoff = (k // K) * CQ

                def _mloop(j, _):
                    vals = plsc.load_gather(buf3, [t3, j * _L + iota, lane_b])
                    o = qoff + j * _L
                    out_buf[pl.ds(o, _L)] = jnp.maximum(
                        out_buf[pl.ds(o, _L)], vals)
                    return 0

                lax.fori_loop(0, CQ // _L, _mloop, 0)

    _fire(0)

    @pl.when(T > 1)
    def _():
        _fire(1)

    def _step(t, _):
        @pl.when(t + 2 < T)
        def _():
            _fire(t + 2)

        _consume(t)
        return 0

    lax.fori_loop(0, T, _step, 0)

    pltpu.sync_copy(out_buf.at[pl.ds(0, CQ)], out_hbm.at[0, b, pl.ds(c0, CQ)])
    pltpu.sync_copy(out_buf.at[pl.ds(CQ, CQ)], out_hbm.at[1, b, pl.ds(c0, CQ)])


def _sc_maxpool_v4(point_features, meta):
    B, C, N = point_features.shape
    CQ = C // 4
    pf2 = point_features.reshape(B * C, N)
    mesh = plsc.VectorSubcoreMesh(core_axis_name="c", subcore_axis_name="s")
    body = functools.partial(_sc_body_v4, B=B, C=C, N=N)
    f = pl.kernel(
        body,
        out_type=jax.ShapeDtypeStruct((2, B, C), jnp.float32),
        mesh=mesh,
        compiler_params=pltpu.CompilerParams(needs_layout_passes=False),
        scratch_types=[
            pltpu.VMEM((6 * K + _L,), jnp.int32),
            pltpu.VMEM((3, CQ, 128), jnp.float32),
            pltpu.VMEM((2 * CQ,), jnp.float32),
            pltpu.SemaphoreType.DMA((3,)),
        ],
    )
    return f(pf2, meta)


def _sc_body(pf_hbm, idx_hbm, out_hbm, nk_v, rows_a, rows_b, rows_c, out_buf,
             sem_a, sem_b, sem_c, *, B, C, N):
    CH = C // 2
    s = lax.axis_index("s")
    c = lax.axis_index("c")
    wid = s * _NC + c
    pair = wid // 2
    q = pair // B
    b = pair % B
    c0 = (wid % 2) * CH

    pltpu.sync_copy(idx_hbm.at[q, b], nk_v)
    iota = lax.iota(jnp.int32, _L)
    nk_lo = nk_v[pl.ds(0, _L)]
    nk_hi = nk_v[pl.ds(_L, _L)]
    rstart = b * C + c0

    def _iloop(j, _):
        out_buf[pl.ds(j * _L, _L)] = jnp.full((_L,), float(NEG), jnp.float32)
        return 0

    lax.fori_loop(0, CH // _L, _iloop, 0)

    nchunks = CH // _RC
    bufs = [(rows_a, sem_a), (rows_b, sem_b), (rows_c, sem_c)]
    nbuf = len(bufs)
    steps = []
    for k in range(K):
        vec = nk_lo if k < _L else nk_hi
        nk = jnp.max(jnp.where(iota == (k % _L), vec, 0))
        n0 = pl.multiple_of((nk // 128) * 128, 128)
        lane = nk % 128
        for h in range(nchunks):
            steps.append((k, h, n0, lane))

    def _fire(i):
        _, h, n0, _ = steps[i]
        buf, sem = bufs[i % nbuf]
        pltpu.make_async_copy(
            pf_hbm.at[pl.ds(rstart + h * _RC, _RC), pl.ds(n0, 128)],
            buf, sem).start()

    def _consume(i):
        _, h, _, lane = steps[i]
        buf, sem = bufs[i % nbuf]
        pltpu.make_async_copy(
            pf_hbm.at[pl.ds(rstart, _RC), pl.ds(0, 128)], buf, sem).wait()
        lane_b = jnp.broadcast_to(lane, (_L,))

        def _mloop(j, _, h=h, buf=buf, lane_b=lane_b):
            pos = j * _L + iota
            vals = plsc.load_gather(buf, [pos, lane_b])
            o = h * _RC + j * _L
            out_buf[pl.ds(o, _L)] = jnp.maximum(out_buf[pl.ds(o, _L)], vals)
            return 0

        lax.fori_loop(0, _RC // _L, _mloop, 0)

    for i in range(nbuf - 1):
        _fire(i)
    for i in range(len(steps)):
        if i + nbuf - 1 < len(steps):
            _fire(i + nbuf - 1)
        _consume(i)

    pltpu.sync_copy(out_buf, out_hbm.at[q, b, pl.ds(c0, CH)])


def _sc_maxpool(point_features, idx):
    B, C, N = point_features.shape
    CH = C // 2
    pf2 = point_features.reshape(B * C, N)
    mesh = plsc.VectorSubcoreMesh(core_axis_name="c", subcore_axis_name="s")
    body = functools.partial(_sc_body, B=B, C=C, N=N)
    f = pl.kernel(
        body,
        out_type=jax.ShapeDtypeStruct((2, B, C), jnp.float32),
        mesh=mesh,
        compiler_params=pltpu.CompilerParams(needs_layout_passes=False),
        scratch_types=[
            pltpu.VMEM((K,), jnp.int32),
            pltpu.VMEM((_RC, 128), jnp.float32),
            pltpu.VMEM((_RC, 128), jnp.float32),
            pltpu.VMEM((_RC, 128), jnp.float32),
            pltpu.VMEM((CH,), jnp.float32),
            pltpu.SemaphoreType.DMA,
            pltpu.SemaphoreType.DMA,
            pltpu.SemaphoreType.DMA,
        ],
    )
    return f(pf2, idx)


def _tcg_body(idx_ref, pf_ref, out_ref, buf, sem):
    n2, B, _ = idx_ref.shape
    C = out_ref.shape[1]
    for q in range(2):
        for b in range(B):
            for k in range(K):
                col = (q * B + b) * K + k
                nk = idx_ref[q, b, k]
                pltpu.make_async_copy(
                    pf_ref.at[b, :, pl.ds(nk, 1)],
                    buf.at[:, pl.ds(col, 1)],
                    sem,
                ).start()
    pltpu.make_async_copy(pf_ref.at[0, :, pl.ds(0, 2 * B * K)], buf, sem).wait()
    for qb in range(2 * B):
        seg = buf[:, pl.ds(qb * K, K)]
        out_ref[pl.ds(qb, 1), :] = jnp.max(seg, axis=1)[None, :]


def _tc_gather_max(point_features, idx):
    B, C, N = point_features.shape
    out = pl.pallas_call(
        _tcg_body,
        in_specs=[
            pl.BlockSpec(memory_space=pltpu.SMEM),
            pl.BlockSpec(memory_space=pltpu.HBM),
        ],
        out_specs=pl.BlockSpec(memory_space=pltpu.VMEM),
        out_shape=jax.ShapeDtypeStruct((2 * B, C), jnp.float32),
        scratch_shapes=[
            pltpu.VMEM((C, 2 * B * K), jnp.float32),
            pltpu.SemaphoreType.DMA,
        ],
    )(idx, point_features)
    return out.reshape(2, B, C)


def _mlp_body(jl_ref, dl_ref, jw1, jb1, jw2, jb2, dw1, db1, dw2, db2,
              fw1, fb1, fw2, fb2, out_ref):
    jl = jl_ref[...]
    dl = dl_ref[...]
    f32 = jnp.float32
    jf = jnp.maximum(jnp.dot(jl, jw1[...], preferred_element_type=f32)
                     + jb1[...], 0.0)
    jf = jnp.dot(jf, jw2[...], preferred_element_type=f32) + jb2[...]
    df = jnp.maximum(jnp.dot(dl, dw1[...], preferred_element_type=f32)
                     + db1[...], 0.0)
    df = jnp.dot(df, dw2[...], preferred_element_type=f32) + db2[...]
    comb = jnp.concatenate([jf, df], axis=1)
    h = jnp.maximum(jnp.dot(comb, fw1[...], preferred_element_type=f32)
                    + fb1[...], 0.0)
    out_ref[...] = jnp.dot(h, fw2[...], preferred_element_type=f32) + fb2[...]


def _mlps(jl, dl, jw1, jb1, jw2, jb2, dw1, db1, dw2, db2, fw1, fb1, fw2, fb2):
    B = jl.shape[0]
    O = fw2.shape[1]
    args = (jl, dl, jw1, jb1[None, :], jw2, jb2[None, :],
            dw1, db1[None, :], dw2, db2[None, :],
            fw1, fb1[None, :], fw2, fb2[None, :])
    return pl.pallas_call(
        _mlp_body,
        out_shape=jax.ShapeDtypeStruct((B, O), jnp.float32),
    )(*args)


def kernel(points_xyz, point_features, joint_origin, drag_point,
           jw1, jb1, jw2, jb2, dw1, db1, dw2, db2, fw1, fb1, fw2, fb2):
    meta = _topk(points_xyz, joint_origin, drag_point)
    loc = _sc_maxpool_v4(point_features, meta)
    return _mlps(loc[0], loc[1], jw1, jb1, jw2, jb2,
                 dw1, db1, dw2, db2, fw1, fb1, fw2, fb2)


# speedup vs baseline: 1.3876x; 1.3876x over previous
import functools

import jax
import jax.numpy as jnp
from jax import lax
from jax.experimental import pallas as pl
from jax.experimental.pallas import tpu as pltpu
from jax.experimental.pallas import tpu_sc as plsc

K = 32
_NC, _NS, _L = 2, 16, 16
NEG = -3.0e38
BIG = 3.0e38


def _topk_body(xyz_ref, q_ref, meta_ref):
    x = xyz_ref[0]
    y = xyz_ref[1]
    z = xyz_ref[2]
    B, N = x.shape
    iota = jax.lax.broadcasted_iota(jnp.int32, (2 * B, N), 1)
    parts = []
    for q in range(2):
        dx = x - q_ref[q, 0]
        dy = y - q_ref[q, 1]
        dz = z - q_ref[q, 2]
        parts.append(dx * dx + dy * dy + dz * dz)
    d2 = jnp.concatenate(parts, axis=0)
    args = []
    for k in range(K):
        m = jnp.min(d2, axis=1, keepdims=True)
        arg = jnp.min(jnp.where(d2 == m, iota, N), axis=1, keepdims=True)
        meta_ref[:, :, pl.ds(k, 1)] = arg.reshape(2, B, 1)
        d2 = jnp.where(iota == arg, BIG, d2)
        args.append(arg)
    blk = [a // 128 for a in args]
    isf = []
    for k in range(K):
        f = jnp.ones_like(blk[0], dtype=jnp.bool_)
        for j in range(k):
            f = jnp.logical_and(f, blk[j] != blk[k])
        isf.append(f)
    uu = []
    run = jnp.zeros_like(blk[0])
    for k in range(K):
        run = run + isf[k].astype(jnp.int32)
        uu.append(run - 1)
    mcnt = run
    for k in range(K):
        pos = jnp.zeros_like(blk[0])
        for j in range(k + 1):
            hit = jnp.logical_and(isf[j], blk[j] == blk[k])
            pos = pos + jnp.where(hit, uu[j], 0)
        meta_ref[:, :, pl.ds(2 * K + k, 1)] = pos.reshape(2, B, 1)
    for i in range(K):
        u = jnp.zeros_like(blk[0])
        for j in range(K):
            hit = jnp.logical_and(isf[j], uu[j] == i)
            u = u + jnp.where(hit, blk[j], 0)
        meta_ref[:, :, pl.ds(K + i, 1)] = u.reshape(2, B, 1)
        meta_ref[:, :, pl.ds(3 * K + i, 1)] = mcnt.reshape(2, B, 1)


def _topk(points_xyz, joint_origin, drag_point):
    B, N, _ = points_xyz.shape
    xyz_t = jnp.moveaxis(points_xyz, -1, 0)
    q = jnp.stack([joint_origin, drag_point], 0)
    q = jnp.moveaxis(q, -1, 1)[..., None]
    return pl.pallas_call(
        _topk_body,
        out_shape=jax.ShapeDtypeStruct((2, B, 4 * K), jnp.int32),
    )(xyz_t, q)


_RC = 256


def _sc_body_v4(pf_hbm, meta_hbm, out_hbm, meta_v, buf3, out_buf, sems,
                *, B, C, N):
    CH = C // 2
    s = lax.axis_index("s")
    c = lax.axis_index("c")
    wid = s * _NC + c
    pair = wid // 2
    q = pair // B
    b = pair % B
    c0 = (wid % 2) * CH

    pltpu.sync_copy(meta_hbm.at[q, b], meta_v)
    iota = lax.iota(jnp.int32, _L)
    nk_lo = meta_v[pl.ds(0, _L)]
    nk_hi = meta_v[pl.ds(_L, _L)]
    ub_lo = meta_v[pl.ds(2 * _L, _L)]
    ub_hi = meta_v[pl.ds(3 * _L, _L)]
    po_lo = meta_v[pl.ds(4 * _L, _L)]
    po_hi = meta_v[pl.ds(5 * _L, _L)]
    mcnt = jnp.max(meta_v[pl.ds(6 * _L, _L)])
    rstart = b * C + c0

    lanes, poss = [], []
    for k in range(K):
        nv = nk_lo if k < _L else nk_hi
        pv = po_lo if k < _L else po_hi
        sel = iota == (k % _L)
        lanes.append(jnp.max(jnp.where(sel, nv, 0)) % 128)
        poss.append(jnp.max(jnp.where(sel, pv, 0)))

    def _iloop(j, _):
        out_buf[pl.ds(j * _L, _L)] = jnp.full((_L,), float(NEG), jnp.float32)
        return 0

    lax.fori_loop(0, CH // _L, _iloop, 0)

    nchunks = CH // _RC
    T = mcnt * nchunks

    def _ub(i):
        lo = jnp.max(jnp.where((iota == (i % _L)) & (i < _L), ub_lo, 0))
        hi = jnp.max(jnp.where((iota == (i % _L)) & (i >= _L), ub_hi, 0))
        return jnp.maximum(lo, hi)

    def _fire(t):
        i = t // nchunks
        h = t % nchunks
        n0 = pl.multiple_of(_ub(i) * 128, 128)
        pltpu.make_async_copy(
            pf_hbm.at[pl.ds(rstart + h * _RC, _RC), pl.ds(n0, 128)],
            buf3.at[t % 3], sems.at[t % 3]).start()

    def _consume(t):
        i = t // nchunks
        h = t % nchunks
        pltpu.make_async_copy(
            pf_hbm.at[pl.ds(rstart, _RC), pl.ds(0, 128)],
            buf3.at[t % 3], sems.at[t % 3]).wait()
        t3 = jnp.broadcast_to(t % 3, (_L,))
        for k in range(K):
            @pl.when(poss[k] == i)
            def _(k=k, t3=t3, h=h):
                lane_b = jnp.broadcast_to(lanes[k], (_L,))

                def _mloop(j, _):
                    vals = plsc.load_gather(buf3, [t3, j * _L + iota, lane_b])
                    o = h * _RC + j * _L
                    out_buf[pl.ds(o, _L)] = jnp.maximum(
                        out_buf[pl.ds(o, _L)], vals)
                    return 0

                lax.fori_loop(0, _RC // _L, _mloop, 0)

    _fire(0)
    _fire(1)

    def _step(t, _):
        @pl.when(t + 2 < T)
        def _():
            _fire(t + 2)

        _consume(t)
        return 0

    lax.fori_loop(0, T, _step, 0)

    pltpu.sync_copy(out_buf, out_hbm.at[q, b, pl.ds(c0, CH)])


def _sc_maxpool_v4(point_features, meta):
    B, C, N = point_features.shape
    CH = C // 2
    pf2 = point_features.reshape(B * C, N)
    mesh = plsc.VectorSubcoreMesh(core_axis_name="c", subcore_axis_name="s")
    body = functools.partial(_sc_body_v4, B=B, C=C, N=N)
    f = pl.kernel(
        body,
        out_type=jax.ShapeDtypeStruct((2, B, C), jnp.float32),
        mesh=mesh,
        compiler_params=pltpu.CompilerParams(needs_layout_passes=False),
        scratch_types=[
            pltpu.VMEM((4 * K,), jnp.int32),
            pltpu.VMEM((3, _RC, 128), jnp.float32),
            pltpu.VMEM((CH,), jnp.float32),
            pltpu.SemaphoreType.DMA((3,)),
        ],
    )
    return f(pf2, meta)


def _mlp_body(jl_ref, dl_ref, jw1, jb1, jw2, jb2, dw1, db1, dw2, db2,
              fw1, fb1, fw2, fb2, out_ref):
    jl = jl_ref[...]
    dl = dl_ref[...]
    f32 = jnp.float32
    jf = jnp.maximum(jnp.dot(jl, jw1[...], preferred_element_type=f32)
                     + jb1[...], 0.0)
    jf = jnp.dot(jf, jw2[...], preferred_element_type=f32) + jb2[...]
    df = jnp.maximum(jnp.dot(dl, dw1[...], preferred_element_type=f32)
                     + db1[...], 0.0)
    df = jnp.dot(df, dw2[...], preferred_element_type=f32) + db2[...]
    comb = jnp.concatenate([jf, df], axis=1)
    h = jnp.maximum(jnp.dot(comb, fw1[...], preferred_element_type=f32)
                    + fb1[...], 0.0)
    out_ref[...] = jnp.dot(h, fw2[...], preferred_element_type=f32) + fb2[...]


def _mlps(jl, dl, jw1, jb1, jw2, jb2, dw1, db1, dw2, db2, fw1, fb1, fw2, fb2):
    B = jl.shape[0]
    O = fw2.shape[1]
    args = (jl, dl, jw1, jb1[None, :], jw2, jb2[None, :],
            dw1, db1[None, :], dw2, db2[None, :],
            fw1, fb1[None, :], fw2, fb2[None, :])
    return pl.pallas_call(
        _mlp_body,
        out_shape=jax.ShapeDtypeStruct((B, O), jnp.float32),
    )(*args)


def kernel(points_xyz, point_features, joint_origin, drag_point,
           jw1, jb1, jw2, jb2, dw1, db1, dw2, db2, fw1, fb1, fw2, fb2):
    meta = _topk(points_xyz, joint_origin, drag_point)
    loc = _sc_maxpool_v4(point_features, meta)
    return _mlps(loc[0], loc[1], jw1, jb1, jw2, jb2,
                 dw1, db1, dw2, db2, fw1, fb1, fw2, fb2)
